# baseline (speedup 1.0000x reference)
import jax
import jax.numpy as jnp
from jax.experimental import pallas as pl
from jax.experimental.pallas import tpu as pltpu

N = 4096
D = 512
K = 31
BM = 512
BISECT_ITERS = 26


def _fused_body(x_ref, w0_ref, w1_ref, out_ref, emb_ref):
    i = pl.program_id(0)

    @pl.when(i == 0)
    def _encode():
        h = x_ref[:] * w0_ref[:]
        h = jnp.maximum(h, 0.0)
        h = h * w1_ref[:]
        s = jnp.sum(h * h, axis=-1, keepdims=True)
        n = jnp.sqrt(s)
        emb_ref[:] = h / jnp.maximum(n, 1e-12)

    rows = emb_ref[pl.ds(i * BM, BM), :]
    a = jax.lax.dot_general(
        rows, emb_ref[:],
        dimension_numbers=(((1,), (1,)), ((), ())),
        preferred_element_type=jnp.float32,
    )

    ones = jnp.ones((N, 128), jnp.bfloat16)

    def body(_, carry):
        lo, hi, clo = carry
        mid = (lo + hi) * 0.5
        sel = jnp.where(a >= mid, 1.0, 0.0).astype(jnp.bfloat16)
        cnt = jax.lax.dot_general(
            sel, ones,
            dimension_numbers=(((1,), (0,)), ((), ())),
            preferred_element_type=jnp.float32,
        )[:, :1]
        ge = cnt >= K
        return (jnp.where(ge, mid, lo),
                jnp.where(ge, hi, mid),
                jnp.where(ge, cnt, clo))

    lo0 = jnp.full((BM, 1), -1.01, jnp.float32)
    hi0 = jnp.full((BM, 1), 1.01, jnp.float32)
    clo0 = jnp.full((BM, 1), float(N), jnp.float32)
    lo, _, clo = jax.lax.fori_loop(0, BISECT_ITERS, body, (lo0, hi0, clo0))
    t = jnp.min(jnp.where(a >= lo, a, 2.0), axis=1, keepdims=True)
    t = jnp.where(clo == K, t, lo)
    out_ref[:] = jnp.where((a >= t) & (a > 0.0), a, 0.0)


@jax.jit
def kernel(x, w0, w1):
    return pl.pallas_call(
        _fused_body,
        grid=(N // BM,),
        in_specs=[
            pl.BlockSpec((N, D), lambda i: (0, 0)),
            pl.BlockSpec((1, D), lambda i: (0, 0)),
            pl.BlockSpec((1, D), lambda i: (0, 0)),
        ],
        out_specs=pl.BlockSpec((BM, N), lambda i: (i, 0)),
        out_shape=jax.ShapeDtypeStruct((N, N), jnp.float32),
        scratch_shapes=[pltpu.VMEM((N, D), jnp.float32)],
    )(x, w0.reshape(1, D), w1.reshape(1, D))


# speedup vs baseline: 1.3001x; 1.3001x over previous
import jax
import jax.numpy as jnp
from jax.experimental import pallas as pl
from jax.experimental.pallas import tpu as pltpu

N = 4096
D = 512
K = 31
BM = 512
BISECT_ITERS = 26


def _fused_body(x_ref, w0_ref, w1_ref, out_ref, emb_ref):
    i = pl.program_id(0)

    @pl.when(i == 0)
    def _encode():
        h = x_ref[:] * w0_ref[:]
        h = jnp.maximum(h, 0.0)
        h = h * w1_ref[:]
        s = jnp.sum(h * h, axis=-1, keepdims=True)
        n = jnp.sqrt(s)
        emb_ref[:] = h / jnp.maximum(n, 1e-12)

    rows = emb_ref[pl.ds(i * BM, BM), :]
    a = jax.lax.dot_general(
        rows, emb_ref[:],
        dimension_numbers=(((1,), (1,)), ((), ())),
        preferred_element_type=jnp.float32,
    )

    def body(_, carry):
        lo, hi, clo = carry
        mid = (lo + hi) * 0.5
        cnt = jnp.sum(jnp.where(a >= mid, 1.0, 0.0), axis=1, keepdims=True)
        ge = cnt >= K
        return (jnp.where(ge, mid, lo),
                jnp.where(ge, hi, mid),
                jnp.where(ge, cnt, clo))

    lo0 = jnp.full((BM, 1), -1.01, jnp.float32)
    hi0 = jnp.full((BM, 1), 1.01, jnp.float32)
    clo0 = jnp.full((BM, 1), float(N), jnp.float32)
    lo, _, clo = jax.lax.fori_loop(0, BISECT_ITERS, body, (lo0, hi0, clo0))
    t = jnp.min(jnp.where(a >= lo, a, 2.0), axis=1, keepdims=True)
    t = jnp.where(clo == K, t, lo)
    out_ref[:] = jnp.where((a >= t) & (a > 0.0), a, 0.0)


@jax.jit
def kernel(x, w0, w1):
    return pl.pallas_call(
        _fused_body,
        grid=(N // BM,),
        in_specs=[
            pl.BlockSpec((N, D), lambda i: (0, 0)),
            pl.BlockSpec((1, D), lambda i: (0, 0)),
            pl.BlockSpec((1, D), lambda i: (0, 0)),
        ],
        out_specs=pl.BlockSpec((BM, N), lambda i: (i, 0)),
        out_shape=jax.ShapeDtypeStruct((N, N), jnp.float32),
        scratch_shapes=[pltpu.VMEM((N, D), jnp.float32)],
    )(x, w0.reshape(1, D), w1.reshape(1, D))


# speedup vs baseline: 1.4732x; 1.1332x over previous
import jax
import jax.numpy as jnp
from jax.experimental import pallas as pl
from jax.experimental.pallas import tpu as pltpu

N = 4096
D = 512
K = 31
BM = 512
BISECT_ITERS = 20


def _fused_body(x_ref, w0_ref, w1_ref, out_ref, emb_ref):
    i = pl.program_id(0)

    @pl.when(i == 0)
    def _encode():
        h = x_ref[:] * w0_ref[:]
        h = jnp.maximum(h, 0.0)
        h = h * w1_ref[:]
        s = jnp.sum(h * h, axis=-1, keepdims=True)
        n = jnp.sqrt(s)
        emb_ref[:] = h / jnp.maximum(n, 1e-12)

    rows = emb_ref[pl.ds(i * BM, BM), :]
    a = jax.lax.dot_general(
        rows, emb_ref[:],
        dimension_numbers=(((1,), (1,)), ((), ())),
        preferred_element_type=jnp.float32,
    )

    def body(_, carry):
        lo, hi, clo = carry
        mid = (lo + hi) * 0.5
        cnt = jnp.sum(jnp.where(a >= mid, 1.0, 0.0), axis=1, keepdims=True)
        ge = cnt >= K
        return (jnp.where(ge, mid, lo),
                jnp.where(ge, hi, mid),
                jnp.where(ge, cnt, clo))

    lo0 = jnp.full((BM, 1), -1.01, jnp.float32)
    hi0 = jnp.full((BM, 1), 1.01, jnp.float32)
    clo0 = jnp.full((BM, 1), float(N), jnp.float32)
    lo, _, clo = jax.lax.fori_loop(0, BISECT_ITERS, body, (lo0, hi0, clo0))
    cand = a >= lo
    m1 = jnp.min(jnp.where(cand, a, 2.0), axis=1, keepdims=True)
    m2 = jnp.min(jnp.where(cand & (a > m1), a, 2.0), axis=1, keepdims=True)
    m3 = jnp.min(jnp.where(cand & (a > m2), a, 2.0), axis=1, keepdims=True)
    t = jnp.where(clo == K, m1,
                  jnp.where(clo == K + 1, m2,
                            jnp.where(clo == K + 2, m3, lo)))
    out_ref[:] = jnp.where((a >= t) & (a > 0.0), a, 0.0)


@jax.jit
def kernel(x, w0, w1):
    return pl.pallas_call(
        _fused_body,
        grid=(N // BM,),
        in_specs=[
            pl.BlockSpec((N, D), lambda i: (0, 0)),
            pl.BlockSpec((1, D), lambda i: (0, 0)),
            pl.BlockSpec((1, D), lambda i: (0, 0)),
        ],
        out_specs=pl.BlockSpec((BM, N), lambda i: (i, 0)),
        out_shape=jax.ShapeDtypeStruct((N, N), jnp.float32),
        scratch_shapes=[pltpu.VMEM((N, D), jnp.float32)],
    )(x, w0.reshape(1, D), w1.reshape(1, D))


# speedup vs baseline: 1.6524x; 1.1216x over previous
import jax
import jax.numpy as jnp
from jax.experimental import pallas as pl
from jax.experimental.pallas import tpu as pltpu

N = 4096
D = 512
K = 31
BM = 512
BISECT_ITERS = 16
N_STRIP = 5


def _fused_body(x_ref, w0_ref, w1_ref, out_ref, emb_ref):
    i = pl.program_id(0)

    @pl.when(i == 0)
    def _encode():
        h = x_ref[:] * w0_ref[:]
        h = jnp.maximum(h, 0.0)
        h = h * w1_ref[:]
        s = jnp.sum(h * h, axis=-1, keepdims=True)
        n = jnp.sqrt(s)
        emb_ref[:] = h / jnp.maximum(n, 1e-12)

    rows = emb_ref[pl.ds(i * BM, BM), :]
    a = jax.lax.dot_general(
        rows, emb_ref[:],
        dimension_numbers=(((1,), (1,)), ((), ())),
        preferred_element_type=jnp.float32,
    )

    def body(_, carry):
        lo, hi, clo = carry
        mid = (lo + hi) * 0.5
        cnt = jnp.sum(jnp.where(a >= mid, 1.0, 0.0), axis=1, keepdims=True)
        ge = cnt >= K
        return (jnp.where(ge, mid, lo),
                jnp.where(ge, hi, mid),
                jnp.where(ge, cnt, clo))

    lo0 = jnp.full((BM, 1), -1.01, jnp.float32)
    hi0 = jnp.full((BM, 1), 1.01, jnp.float32)
    clo0 = jnp.full((BM, 1), float(N), jnp.float32)
    lo, _, clo = jax.lax.fori_loop(0, BISECT_ITERS, body, (lo0, hi0, clo0))
    candf = jnp.where(a >= lo, a, 2.0)
    t = jnp.min(candf, axis=1, keepdims=True)
    excess = clo - K
    for _ in range(N_STRIP):
        nxt = jnp.min(jnp.where(candf > t, candf, 2.0), axis=1, keepdims=True)
        t = jnp.where(excess >= 1.0, nxt, t)
        excess = excess - 1.0
    t = jnp.where(clo > K + N_STRIP, lo, t)
    t = jnp.maximum(t, 1.18e-38)
    out_ref[:] = jnp.where(a >= t, a, 0.0)


@jax.jit
def kernel(x, w0, w1):
    return pl.pallas_call(
        _fused_body,
        grid=(N // BM,),
        in_specs=[
            pl.BlockSpec((N, D), lambda i: (0, 0)),
            pl.BlockSpec((1, D), lambda i: (0, 0)),
            pl.BlockSpec((1, D), lambda i: (0, 0)),
        ],
        out_specs=pl.BlockSpec((BM, N), lambda i: (i, 0)),
        out_shape=jax.ShapeDtypeStruct((N, N), jnp.float32),
        scratch_shapes=[pltpu.VMEM((N, D), jnp.float32)],
    )(x, w0.reshape(1, D), w1.reshape(1, D))
